# baseline (speedup 1.0000x reference)
import jax
import jax.numpy as jnp
from jax.experimental import pallas as pl
from jax.experimental.pallas import tpu as pltpu


_BLOCK_ROWS = 4096


def _copy_kernel(src_ref, dst_ref, sem):
    i = pl.program_id(0)
    copy = pltpu.make_async_copy(
        src_ref.at[pl.ds(i * _BLOCK_ROWS, _BLOCK_ROWS)], dst_ref, sem
    )
    copy.start()
    copy.wait()


def kernel(prototypes):
    rows, feat = prototypes.shape
    return pl.pallas_call(
        _copy_kernel,
        out_shape=jax.ShapeDtypeStruct(prototypes.shape, prototypes.dtype),
        grid=(rows // _BLOCK_ROWS,),
        in_specs=[pl.BlockSpec(memory_space=pl.ANY)],
        out_specs=pl.BlockSpec((_BLOCK_ROWS, feat), lambda i: (i, 0)),
        scratch_shapes=[pltpu.SemaphoreType.DMA],
    )(prototypes)


# speedup vs baseline: 1.0892x; 1.0892x over previous
import functools

import jax
import jax.numpy as jnp
from jax import lax
from jax.experimental import pallas as pl
from jax.experimental.pallas import tpu as pltpu

_ROWS = 8192
_FEAT = 256
_NCORES = 2
_PER_CORE = _ROWS // _NCORES
_CHUNK = _PER_CORE // 2


def _copy_body(src_ref, dst_ref, buf, in_sems, out_sems):
    c = lax.axis_index("x")
    base = c * _PER_CORE
    ins, outs = [], []
    for i in range(2):
        o = base + i * _CHUNK
        cp = pltpu.make_async_copy(
            src_ref.at[pl.ds(o, _CHUNK)], buf.at[i], in_sems.at[i]
        )
        cp.start()
        ins.append(cp)
        outs.append(
            pltpu.make_async_copy(
                buf.at[i], dst_ref.at[pl.ds(o, _CHUNK)], out_sems.at[i]
            )
        )
    for i in range(2):
        ins[i].wait()
        outs[i].start()
    for cp in outs:
        cp.wait()


def kernel(prototypes):
    mesh = pltpu.create_tensorcore_mesh("x", num_cores=_NCORES)
    k = functools.partial(
        pl.kernel,
        mesh=mesh,
        out_type=jax.ShapeDtypeStruct((_ROWS, _FEAT), jnp.float32),
        scratch_types=[
            pltpu.VMEM((2, _CHUNK, _FEAT), jnp.float32),
            pltpu.SemaphoreType.DMA((2,)),
            pltpu.SemaphoreType.DMA((2,)),
        ],
    )(_copy_body)
    return k(prototypes)
